# baseline (speedup 1.0000x reference)
import functools

import jax
import jax.numpy as jnp
from jax import lax
from jax.experimental import pallas as pl
from jax.experimental.pallas import tpu as pltpu
from jax.experimental.pallas import tpu_sc as plsc

NN = 10000
EE = 320000
DD = 128
HH = 128
CC = 40
NP = 10240
CW = 128
NC = 2
NS = 16
NW = NC * NS
CHUNK = 128
CPW = 80
EPAD = NW * CPW * CHUNK
RPT = NP // NS

_F32 = jnp.float32

def _sc_mesh():
    return plsc.VectorSubcoreMesh(
        core_axis_name="c", subcore_axis_name="s",
        num_cores=NC, num_subcores=NS)


SEC = 8
CPW_FAST = 160
CPW_SLOW = CPW * NC - CPW_FAST
FAST_CORE = 0
NSEC_MAX = CPW_FAST // SEC


def _prop_body(width, xs_hbm, edges_hbm, out_hbm,
               sec_a, sec_b, buf_a, buf_b,
               isem_a, isem_b, gsem_a, gsem_b, acc_sh):
    c = lax.axis_index("c")
    s = lax.axis_index("s")
    is_fast = c == FAST_CORE
    base = lax.select(is_fast, s * CPW_FAST,
                      NS * CPW_FAST + s * CPW_SLOW)
    nsec = lax.select(is_fast, CPW_FAST // SEC, CPW_SLOW // SEC)
    secs = (sec_a, sec_b)
    isems = (isem_a, isem_b)

    def sec_load(si, buf, sem):
        return pltpu.async_copy(
            edges_hbm.at[pl.ds(base + si * SEC, SEC)], buf, sem)

    def sec_wait(si, buf, sem):
        pltpu.make_async_copy(
            edges_hbm.at[pl.ds(base + si * SEC, SEC)], buf, sem).wait()

    def gather(sec, k, buf, sem):
        return pltpu.async_copy(xs_hbm.at[sec.at[k, 0]], buf, sem)

    def gwait(sec, k, buf, sem):
        pltpu.make_async_copy(xs_hbm.at[sec.at[k, 0]], buf, sem).wait()

    def scatter(sec, k, buf):
        pltpu.sync_copy(buf, acc_sh.at[sec.at[k, 1]], add=True)

    def zrow(i, carry):
        for u in range(width // 16):
            buf_a[i, pl.ds(u * 16, 16)] = jnp.zeros((16,), _F32)
        return carry

    lax.fori_loop(0, CHUNK, zrow, 0)
    for r in range(RPT // CHUNK):
        pltpu.sync_copy(buf_a, acc_sh.at[pl.ds(s * RPT + r * CHUNK, CHUNK)])

    @pl.when(nsec > 0)
    def _():
        sec_load(0, sec_a, isem_a)

    @pl.when(nsec > 1)
    def _():
        sec_load(1, sec_b, isem_b)

    plsc.subcore_barrier()

    def run_section(si, sec, isem):
        sec_wait(si, sec, isem)
        gather(sec, 0, buf_a, gsem_a)
        gather(sec, 1, buf_b, gsem_b)

        def body(t, carry):
            k0 = 2 * t
            gwait(sec, k0, buf_a, gsem_a)
            scatter(sec, k0, buf_a)
            gather(sec, k0 + 2, buf_a, gsem_a)
            gwait(sec, k0 + 1, buf_b, gsem_b)
            scatter(sec, k0 + 1, buf_b)
            gather(sec, k0 + 3, buf_b, gsem_b)
            return carry

        lax.fori_loop(0, SEC // 2 - 1, body, 0)
        gwait(sec, SEC - 2, buf_a, gsem_a)
        scatter(sec, SEC - 2, buf_a)
        gwait(sec, SEC - 1, buf_b, gsem_b)
        scatter(sec, SEC - 1, buf_b)

        @pl.when(si + 2 < nsec)
        def _():
            sec_load(si + 2, sec, isem)

    def sec_step(si, carry):
        @pl.when(si % 2 == 0)
        def _():
            run_section(si, sec_a, isem_a)

        @pl.when(si % 2 == 1)
        def _():
            run_section(si, sec_b, isem_b)

        return carry

    lax.fori_loop(0, nsec, sec_step, 0)

    plsc.subcore_barrier()
    pltpu.sync_copy(acc_sh.at[pl.ds(s * RPT, RPT)],
                    out_hbm.at[c, pl.ds(s * RPT, RPT)])


@functools.lru_cache(maxsize=None)
def _make_prop(width):
    return pl.kernel(
        functools.partial(_prop_body, width),
        out_type=jax.ShapeDtypeStruct((NC, NP, width), _F32),
        mesh=_sc_mesh(),
        scratch_types=[
            pltpu.VMEM((SEC, 2, CHUNK), jnp.int32),
            pltpu.VMEM((SEC, 2, CHUNK), jnp.int32),
            pltpu.VMEM((CHUNK, width), _F32),
            pltpu.VMEM((CHUNK, width), _F32),
            pltpu.SemaphoreType.DMA,
            pltpu.SemaphoreType.DMA,
            pltpu.SemaphoreType.DMA,
            pltpu.SemaphoreType.DMA,
            pltpu.VMEM_SHARED((NP, width), _F32),
        ],
        name=f"sc_gcn_prop_w{width}",
    )


def _prop128(xs, edges3):
    return _make_prop(HH)(xs, edges3)


def _deg_body(dst_hbm, z_hbm, out_hbm, dst_v, ones_v, acc_sh):
    c = lax.axis_index("c")
    s = lax.axis_index("s")
    wid = s * NC + c
    pltpu.sync_copy(z_hbm.at[pl.ds(s * RPT, RPT)],
                    acc_sh.at[pl.ds(s * RPT, RPT)])
    pltpu.sync_copy(dst_hbm.at[pl.ds(wid * CPW, CPW)], dst_v)
    for i in range(CHUNK // 16):
        ones_v[pl.ds(i * 16, 16)] = jnp.full((16,), 1.0, _F32)
    plsc.subcore_barrier()

    def body(j, carry):
        pltpu.sync_copy(ones_v, acc_sh.at[dst_v.at[j]], add=True)
        return carry

    lax.fori_loop(0, CPW, body, 0)
    plsc.subcore_barrier()
    pltpu.sync_copy(acc_sh.at[pl.ds(s * RPT, RPT)],
                    out_hbm.at[c, pl.ds(s * RPT, RPT)])


@functools.lru_cache(maxsize=None)
def _make_deg():
    return pl.kernel(
        _deg_body,
        out_type=jax.ShapeDtypeStruct((NC, NP), _F32),
        mesh=_sc_mesh(),
        scratch_types=[
            pltpu.VMEM((CPW, CHUNK), jnp.int32),
            pltpu.VMEM((CHUNK,), _F32),
            pltpu.VMEM_SHARED((NP,), _F32),
        ],
        name="sc_gcn_deg",
    )


def _deg_counts(dst, zeros):
    return _make_deg()(dst, zeros)


RB = 512
RB2 = 400
_HIGH = lax.Precision.HIGHEST


def _dinv_body(a_ref, o_ref):
    deg = a_ref[0] + a_ref[1] + 1.0
    o_ref[...] = lax.rsqrt(deg)


def _dinv(degp):
    return pl.pallas_call(
        _dinv_body,
        out_shape=jax.ShapeDtypeStruct((NP, 1), _F32),
    )(degp)


def _mm_scale_body(x_ref, w_ref, dinv_ref, o_ref):
    y = jnp.dot(x_ref[...], w_ref[...],
                preferred_element_type=_F32, precision=_HIGH)
    o_ref[...] = y * dinv_ref[...]


def _mm_scale(x, w, dinv):
    width = w.shape[1]
    return pl.pallas_call(
        _mm_scale_body,
        grid=(NP // RB,),
        in_specs=[
            pl.BlockSpec((RB, x.shape[1]), lambda i: (i, 0)),
            pl.BlockSpec((x.shape[1], width), lambda i: (0, 0)),
            pl.BlockSpec((RB, 1), lambda i: (i, 0)),
        ],
        out_specs=pl.BlockSpec((RB, width), lambda i: (i, 0)),
        out_shape=jax.ShapeDtypeStruct((NP, width), _F32),
    )(x, w, dinv)


def _combine_stats_body(acc_ref, xs_ref, dinv_ref, b_ref, t_ref, st_ref):
    i = pl.program_id(0)
    t = dinv_ref[...] * (acc_ref[0] + acc_ref[1] + xs_ref[...]) + b_ref[...]
    t_ref[...] = t
    rowid = lax.broadcasted_iota(jnp.int32, t.shape, 0) + i * RB
    tm = jnp.where(rowid < NN, t, 0.0)

    @pl.when(i == 0)
    def _():
        st_ref[...] = jnp.zeros_like(st_ref)

    st_ref[0:1, :] = st_ref[0:1, :] + jnp.sum(tm, axis=0, keepdims=True)
    st_ref[1:2, :] = st_ref[1:2, :] + jnp.sum(tm * tm, axis=0, keepdims=True)


def _combine_stats(accp, xs, dinv, b):
    width = xs.shape[1]
    return pl.pallas_call(
        _combine_stats_body,
        grid=(NP // RB,),
        in_specs=[
            pl.BlockSpec((NC, RB, width), lambda i: (0, i, 0)),
            pl.BlockSpec((RB, width), lambda i: (i, 0)),
            pl.BlockSpec((RB, 1), lambda i: (i, 0)),
            pl.BlockSpec((1, width), lambda i: (0, 0)),
        ],
        out_specs=[
            pl.BlockSpec((RB, width), lambda i: (i, 0)),
            pl.BlockSpec((8, width), lambda i: (0, 0)),
        ],
        out_shape=[
            jax.ShapeDtypeStruct((NP, width), _F32),
            jax.ShapeDtypeStruct((8, width), _F32),
        ],
    )(accp, xs, dinv, b)


def _bn_mm_body(t_ref, st_ref, g_ref, be_ref, w_ref, dinv_ref, o_ref):
    m = st_ref[0:1, :] * (1.0 / NN)
    var = st_ref[1:2, :] * (1.0 / NN) - m * m
    inv = lax.rsqrt(var + 1e-5)
    h = jnp.maximum((t_ref[...] - m) * inv * g_ref[...] + be_ref[...], 0.0)
    y = jnp.dot(h, w_ref[...], preferred_element_type=_F32, precision=_HIGH)
    o_ref[...] = y * dinv_ref[...]


def _bn_mm_scale(t, st, g, be, w, dinv):
    width = w.shape[1]
    return pl.pallas_call(
        _bn_mm_body,
        grid=(NP // RB,),
        in_specs=[
            pl.BlockSpec((RB, t.shape[1]), lambda i: (i, 0)),
            pl.BlockSpec((8, t.shape[1]), lambda i: (0, 0)),
            pl.BlockSpec((1, t.shape[1]), lambda i: (0, 0)),
            pl.BlockSpec((1, t.shape[1]), lambda i: (0, 0)),
            pl.BlockSpec((t.shape[1], width), lambda i: (0, 0)),
            pl.BlockSpec((RB, 1), lambda i: (i, 0)),
        ],
        out_specs=pl.BlockSpec((RB, width), lambda i: (i, 0)),
        out_shape=jax.ShapeDtypeStruct((NP, width), _F32),
    )(t, st, g, be, w, dinv)


def _final_body(acc_ref, xs_ref, dinv_ref, b_ref, lsm_ref, h_ref):
    h3 = dinv_ref[...] * (acc_ref[0] + acc_ref[1] + xs_ref[...]) + b_ref[...]
    colid = lax.broadcasted_iota(jnp.int32, h3.shape, 1)
    mask = colid < CC
    mx = jnp.max(jnp.where(mask, h3, -jnp.inf), axis=1, keepdims=True)
    e = jnp.where(mask, jnp.exp(h3 - mx), 0.0)
    lsm = h3 - mx - jnp.log(jnp.sum(e, axis=1, keepdims=True))
    lsm_ref[...] = lsm[:, :CC]
    h_ref[...] = h3[:, :CC]


def _final(accp, xs, dinv, b):
    return pl.pallas_call(
        _final_body,
        grid=(NN // RB2,),
        in_specs=[
            pl.BlockSpec((NC, RB2, CW), lambda i: (0, i, 0)),
            pl.BlockSpec((RB2, CW), lambda i: (i, 0)),
            pl.BlockSpec((RB2, 1), lambda i: (i, 0)),
            pl.BlockSpec((1, CW), lambda i: (0, 0)),
        ],
        out_specs=[
            pl.BlockSpec((RB2, CC), lambda i: (i, 0)),
            pl.BlockSpec((RB2, CC), lambda i: (i, 0)),
        ],
        out_shape=[
            jax.ShapeDtypeStruct((NN, CC), _F32),
            jax.ShapeDtypeStruct((NN, CC), _F32),
        ],
    )(accp, xs, dinv, b)


def kernel(features, adj_t, v_sensitive, v_insensitive,
           W1, b1, gamma1, beta1, W2, b2, gamma2, beta2, W3, b3):
    Xp = jnp.zeros((NP, DD), _F32).at[:NN].set(features)
    pad = jnp.full((EPAD - EE,), NN, jnp.int32)
    src = jnp.concatenate([adj_t[0], pad]).reshape(EPAD // CHUNK, CHUNK)
    dst = jnp.concatenate([adj_t[1], pad]).reshape(EPAD // CHUNK, CHUNK)
    edges3 = jnp.stack([src, dst], axis=1)
    W3p = jnp.zeros((HH, CW), _F32).at[:, :CC].set(W3)
    b3p = jnp.zeros((1, CW), _F32).at[:, :CC].set(b3)

    degp = _deg_counts(dst, jnp.zeros((NP,), _F32))
    dinv = _dinv(degp.reshape(NC, NP, 1))

    xs1 = _mm_scale(Xp, W1, dinv)
    acc1 = _prop128(xs1, edges3)
    t1, st1 = _combine_stats(acc1, xs1, dinv, b1.reshape(1, HH))
    xs2 = _bn_mm_scale(t1, st1, gamma1.reshape(1, HH), beta1.reshape(1, HH),
                       W2, dinv)
    acc2 = _prop128(xs2, edges3)
    t2, st2 = _combine_stats(acc2, xs2, dinv, b2.reshape(1, HH))
    xs3 = _bn_mm_scale(t2, st2, gamma2.reshape(1, HH), beta2.reshape(1, HH),
                       W3p, dinv)
    acc3 = _prop128(xs3, edges3)
    lsm, h3 = _final(acc3, xs3, dinv, b3p)
    return (lsm, h3)


# speedup vs baseline: 1.5489x; 1.5489x over previous
import functools

import jax
import jax.numpy as jnp
from jax import lax
from jax.experimental import pallas as pl
from jax.experimental.pallas import tpu as pltpu
from jax.experimental.pallas import tpu_sc as plsc

NN = 10000
EE = 320000
DD = 128
HH = 128
CC = 40
NP = 10240
CW = 128
NC = 2
NS = 16
NW = NC * NS
CHUNK = 128
CPW = 80
EPAD = NW * CPW * CHUNK
RPT = NP // NS

_F32 = jnp.float32

def _sc_mesh():
    return plsc.VectorSubcoreMesh(
        core_axis_name="c", subcore_axis_name="s",
        num_cores=NC, num_subcores=NS)


SEC = 8
CPW_FAST = 152
CPW_SLOW = CPW * NC - CPW_FAST
FAST_CORE = 0
NSEC_MAX = CPW_FAST // SEC


def _prop_body(width, xs_hbm, edges_hbm, out_hbm,
               sec_a, sec_b, buf_a, buf_b,
               isem_a, isem_b, gsem_a, gsem_b, acc_sh):
    c = lax.axis_index("c")
    s = lax.axis_index("s")
    is_fast = c == FAST_CORE
    base = lax.select(is_fast, s * CPW_FAST,
                      NS * CPW_FAST + s * CPW_SLOW)
    nsec = lax.select(is_fast, CPW_FAST // SEC, CPW_SLOW // SEC)
    secs = (sec_a, sec_b)
    isems = (isem_a, isem_b)

    def sec_load(si, buf, sem):
        return pltpu.async_copy(
            edges_hbm.at[pl.ds(base + si * SEC, SEC)], buf, sem)

    def sec_wait(si, buf, sem):
        pltpu.make_async_copy(
            edges_hbm.at[pl.ds(base + si * SEC, SEC)], buf, sem).wait()

    def gather(sec, k, buf, sem):
        return pltpu.async_copy(xs_hbm.at[sec.at[k, 0]], buf, sem)

    def gwait(sec, k, buf, sem):
        pltpu.make_async_copy(xs_hbm.at[sec.at[k, 0]], buf, sem).wait()

    def scatter(sec, k, buf):
        pltpu.sync_copy(buf, acc_sh.at[sec.at[k, 1]], add=True)

    def zrow(i, carry):
        for u in range(width // 16):
            buf_a[i, pl.ds(u * 16, 16)] = jnp.zeros((16,), _F32)
        return carry

    lax.fori_loop(0, CHUNK, zrow, 0)
    for r in range(RPT // CHUNK):
        pltpu.sync_copy(buf_a, acc_sh.at[pl.ds(s * RPT + r * CHUNK, CHUNK)])

    @pl.when(nsec > 0)
    def _():
        sec_load(0, sec_a, isem_a)

    @pl.when(nsec > 1)
    def _():
        sec_load(1, sec_b, isem_b)

    plsc.subcore_barrier()

    for si in range(NSEC_MAX):
        sec, isem = secs[si % 2], isems[si % 2]

        @pl.when(si < nsec)
        def _(sec=sec, isem=isem, si=si):
            sec_wait(si, sec, isem)
            gather(sec, 0, buf_a, gsem_a)
            gather(sec, 1, buf_b, gsem_b)

            def body(t, carry):
                k0 = 2 * t
                gwait(sec, k0, buf_a, gsem_a)
                scatter(sec, k0, buf_a)
                gather(sec, k0 + 2, buf_a, gsem_a)
                gwait(sec, k0 + 1, buf_b, gsem_b)
                scatter(sec, k0 + 1, buf_b)
                gather(sec, k0 + 3, buf_b, gsem_b)
                return carry

            lax.fori_loop(0, SEC // 2 - 1, body, 0)
            gwait(sec, SEC - 2, buf_a, gsem_a)
            scatter(sec, SEC - 2, buf_a)
            gwait(sec, SEC - 1, buf_b, gsem_b)
            scatter(sec, SEC - 1, buf_b)

            @pl.when(si + 2 < nsec)
            def _():
                sec_load(si + 2, sec, isem)

    plsc.subcore_barrier()
    pltpu.sync_copy(acc_sh.at[pl.ds(s * RPT, RPT)],
                    out_hbm.at[c, pl.ds(s * RPT, RPT)])


@functools.lru_cache(maxsize=None)
def _make_prop(width):
    return pl.kernel(
        functools.partial(_prop_body, width),
        out_type=jax.ShapeDtypeStruct((NC, NP, width), _F32),
        mesh=_sc_mesh(),
        scratch_types=[
            pltpu.VMEM((SEC, 2, CHUNK), jnp.int32),
            pltpu.VMEM((SEC, 2, CHUNK), jnp.int32),
            pltpu.VMEM((CHUNK, width), _F32),
            pltpu.VMEM((CHUNK, width), _F32),
            pltpu.SemaphoreType.DMA,
            pltpu.SemaphoreType.DMA,
            pltpu.SemaphoreType.DMA,
            pltpu.SemaphoreType.DMA,
            pltpu.VMEM_SHARED((NP, width), _F32),
        ],
        name=f"sc_gcn_prop_w{width}",
    )


def _prop128(xs, edges3):
    return _make_prop(HH)(xs, edges3)


def _deg_body(dst_hbm, z_hbm, out_hbm, dst_v, ones_v, acc_sh):
    c = lax.axis_index("c")
    s = lax.axis_index("s")
    wid = s * NC + c
    pltpu.sync_copy(z_hbm.at[pl.ds(s * RPT, RPT)],
                    acc_sh.at[pl.ds(s * RPT, RPT)])
    pltpu.sync_copy(dst_hbm.at[pl.ds(wid * CPW, CPW)], dst_v)
    for i in range(CHUNK // 16):
        ones_v[pl.ds(i * 16, 16)] = jnp.full((16,), 1.0, _F32)
    plsc.subcore_barrier()

    def body(j, carry):
        pltpu.sync_copy(ones_v, acc_sh.at[dst_v.at[j]], add=True)
        return carry

    lax.fori_loop(0, CPW, body, 0)
    plsc.subcore_barrier()
    pltpu.sync_copy(acc_sh.at[pl.ds(s * RPT, RPT)],
                    out_hbm.at[c, pl.ds(s * RPT, RPT)])


@functools.lru_cache(maxsize=None)
def _make_deg():
    return pl.kernel(
        _deg_body,
        out_type=jax.ShapeDtypeStruct((NC, NP), _F32),
        mesh=_sc_mesh(),
        scratch_types=[
            pltpu.VMEM((CPW, CHUNK), jnp.int32),
            pltpu.VMEM((CHUNK,), _F32),
            pltpu.VMEM_SHARED((NP,), _F32),
        ],
        name="sc_gcn_deg",
    )


def _deg_counts(dst, zeros):
    return _make_deg()(dst, zeros)


RB = 512
RB2 = 400
_HIGH = lax.Precision.HIGHEST


def _dinv_body(a_ref, o_ref):
    deg = a_ref[0] + a_ref[1] + 1.0
    o_ref[...] = lax.rsqrt(deg)


def _dinv(degp):
    return pl.pallas_call(
        _dinv_body,
        out_shape=jax.ShapeDtypeStruct((NP, 1), _F32),
    )(degp)


def _mm_scale_body(x_ref, w_ref, dinv_ref, o_ref):
    y = jnp.dot(x_ref[...], w_ref[...],
                preferred_element_type=_F32, precision=_HIGH)
    o_ref[...] = y * dinv_ref[...]


def _mm_scale(x, w, dinv):
    width = w.shape[1]
    return pl.pallas_call(
        _mm_scale_body,
        grid=(NP // RB,),
        in_specs=[
            pl.BlockSpec((RB, x.shape[1]), lambda i: (i, 0)),
            pl.BlockSpec((x.shape[1], width), lambda i: (0, 0)),
            pl.BlockSpec((RB, 1), lambda i: (i, 0)),
        ],
        out_specs=pl.BlockSpec((RB, width), lambda i: (i, 0)),
        out_shape=jax.ShapeDtypeStruct((NP, width), _F32),
    )(x, w, dinv)


def _combine_stats_body(acc_ref, xs_ref, dinv_ref, b_ref, t_ref, st_ref):
    i = pl.program_id(0)
    t = dinv_ref[...] * (acc_ref[0] + acc_ref[1] + xs_ref[...]) + b_ref[...]
    t_ref[...] = t
    rowid = lax.broadcasted_iota(jnp.int32, t.shape, 0) + i * RB
    tm = jnp.where(rowid < NN, t, 0.0)

    @pl.when(i == 0)
    def _():
        st_ref[...] = jnp.zeros_like(st_ref)

    st_ref[0:1, :] = st_ref[0:1, :] + jnp.sum(tm, axis=0, keepdims=True)
    st_ref[1:2, :] = st_ref[1:2, :] + jnp.sum(tm * tm, axis=0, keepdims=True)


def _combine_stats(accp, xs, dinv, b):
    width = xs.shape[1]
    return pl.pallas_call(
        _combine_stats_body,
        grid=(NP // RB,),
        in_specs=[
            pl.BlockSpec((NC, RB, width), lambda i: (0, i, 0)),
            pl.BlockSpec((RB, width), lambda i: (i, 0)),
            pl.BlockSpec((RB, 1), lambda i: (i, 0)),
            pl.BlockSpec((1, width), lambda i: (0, 0)),
        ],
        out_specs=[
            pl.BlockSpec((RB, width), lambda i: (i, 0)),
            pl.BlockSpec((8, width), lambda i: (0, 0)),
        ],
        out_shape=[
            jax.ShapeDtypeStruct((NP, width), _F32),
            jax.ShapeDtypeStruct((8, width), _F32),
        ],
    )(accp, xs, dinv, b)


def _bn_mm_body(t_ref, st_ref, g_ref, be_ref, w_ref, dinv_ref, o_ref):
    m = st_ref[0:1, :] * (1.0 / NN)
    var = st_ref[1:2, :] * (1.0 / NN) - m * m
    inv = lax.rsqrt(var + 1e-5)
    h = jnp.maximum((t_ref[...] - m) * inv * g_ref[...] + be_ref[...], 0.0)
    y = jnp.dot(h, w_ref[...], preferred_element_type=_F32, precision=_HIGH)
    o_ref[...] = y * dinv_ref[...]


def _bn_mm_scale(t, st, g, be, w, dinv):
    width = w.shape[1]
    return pl.pallas_call(
        _bn_mm_body,
        grid=(NP // RB,),
        in_specs=[
            pl.BlockSpec((RB, t.shape[1]), lambda i: (i, 0)),
            pl.BlockSpec((8, t.shape[1]), lambda i: (0, 0)),
            pl.BlockSpec((1, t.shape[1]), lambda i: (0, 0)),
            pl.BlockSpec((1, t.shape[1]), lambda i: (0, 0)),
            pl.BlockSpec((t.shape[1], width), lambda i: (0, 0)),
            pl.BlockSpec((RB, 1), lambda i: (i, 0)),
        ],
        out_specs=pl.BlockSpec((RB, width), lambda i: (i, 0)),
        out_shape=jax.ShapeDtypeStruct((NP, width), _F32),
    )(t, st, g, be, w, dinv)


def _final_body(acc_ref, xs_ref, dinv_ref, b_ref, lsm_ref, h_ref):
    h3 = dinv_ref[...] * (acc_ref[0] + acc_ref[1] + xs_ref[...]) + b_ref[...]
    colid = lax.broadcasted_iota(jnp.int32, h3.shape, 1)
    mask = colid < CC
    mx = jnp.max(jnp.where(mask, h3, -jnp.inf), axis=1, keepdims=True)
    e = jnp.where(mask, jnp.exp(h3 - mx), 0.0)
    lsm = h3 - mx - jnp.log(jnp.sum(e, axis=1, keepdims=True))
    lsm_ref[...] = lsm[:, :CC]
    h_ref[...] = h3[:, :CC]


def _final(accp, xs, dinv, b):
    return pl.pallas_call(
        _final_body,
        grid=(NN // RB2,),
        in_specs=[
            pl.BlockSpec((NC, RB2, CW), lambda i: (0, i, 0)),
            pl.BlockSpec((RB2, CW), lambda i: (i, 0)),
            pl.BlockSpec((RB2, 1), lambda i: (i, 0)),
            pl.BlockSpec((1, CW), lambda i: (0, 0)),
        ],
        out_specs=[
            pl.BlockSpec((RB2, CC), lambda i: (i, 0)),
            pl.BlockSpec((RB2, CC), lambda i: (i, 0)),
        ],
        out_shape=[
            jax.ShapeDtypeStruct((NN, CC), _F32),
            jax.ShapeDtypeStruct((NN, CC), _F32),
        ],
    )(accp, xs, dinv, b)


def kernel(features, adj_t, v_sensitive, v_insensitive,
           W1, b1, gamma1, beta1, W2, b2, gamma2, beta2, W3, b3):
    Xp = jnp.zeros((NP, DD), _F32).at[:NN].set(features)
    pad = jnp.full((EPAD - EE,), NN, jnp.int32)
    src = jnp.concatenate([adj_t[0], pad]).reshape(EPAD // CHUNK, CHUNK)
    dst = jnp.concatenate([adj_t[1], pad]).reshape(EPAD // CHUNK, CHUNK)
    edges3 = jnp.stack([src, dst], axis=1)
    W3p = jnp.zeros((HH, CW), _F32).at[:, :CC].set(W3)
    b3p = jnp.zeros((1, CW), _F32).at[:, :CC].set(b3)

    degp = _deg_counts(dst, jnp.zeros((NP,), _F32))
    dinv = _dinv(degp.reshape(NC, NP, 1))

    xs1 = _mm_scale(Xp, W1, dinv)
    acc1 = _prop128(xs1, edges3)
    t1, st1 = _combine_stats(acc1, xs1, dinv, b1.reshape(1, HH))
    xs2 = _bn_mm_scale(t1, st1, gamma1.reshape(1, HH), beta1.reshape(1, HH),
                       W2, dinv)
    acc2 = _prop128(xs2, edges3)
    t2, st2 = _combine_stats(acc2, xs2, dinv, b2.reshape(1, HH))
    xs3 = _bn_mm_scale(t2, st2, gamma2.reshape(1, HH), beta2.reshape(1, HH),
                       W3p, dinv)
    acc3 = _prop128(xs3, edges3)
    lsm, h3 = _final(acc3, xs3, dinv, b3p)
    return (lsm, h3)
